# baseline (speedup 1.0000x reference)
import functools
import jax
import jax.numpy as jnp
import numpy as np
from jax import lax
from jax.experimental import pallas as pl
from jax.experimental.pallas import tpu as pltpu
from jax.experimental.pallas import tpu_sc as plsc

N_NODES = 10000
N_EDGES = 320000
IN_DIM = 128
HIDDEN = 128
OUT_DIM = 32
D_A = 14
DP = 16
NNZ = 64

NC = 2
NS = 16
NW = NC * NS
EW = N_EDGES // NW
C = 80
CHUNKS = EW // C
GC = 25
GE = GC * C
NG = CHUNKS // GC


def _gather_body(h_hbm, src_hbm, tgt_hbm, gs_hbm, gt_hbm,
                 idxbuf, rows, gsem, osem0, osem1):
    c = lax.axis_index("c")
    s = lax.axis_index("s")
    wid = s * NC + c
    ebase = wid * EW
    for idx_hbm, out_hbm in ((src_hbm, gs_hbm), (tgt_hbm, gt_hbm)):
        pltpu.sync_copy(idx_hbm.at[wid], idxbuf)
        osems = (osem0, osem1)
        descs = [None] * NG
        for g in range(NG):
            b = g & 1
            if g >= 2:
                descs[g - 2].wait()

            def fire(j, carry, g=g, b=b):
                pltpu.async_copy(h_hbm.at[idxbuf.at[g * GC + j]],
                                 rows.at[b, pl.ds(j * C, C)], gsem)
                return carry

            lax.fori_loop(0, GC, fire, 0)
            pltpu.make_async_copy(out_hbm.at[pl.ds(ebase + g * GE, GE)],
                                  rows.at[b], gsem).wait()
            descs[g] = pltpu.async_copy(
                rows.at[b], out_hbm.at[pl.ds(ebase + g * GE, GE)], osems[b])
        descs[NG - 2].wait()
        descs[NG - 1].wait()


def _sc_gather(h, src2d, tgt2d):
    mesh = plsc.VectorSubcoreMesh(core_axis_name="c", subcore_axis_name="s",
                                  num_cores=NC, num_subcores=NS)
    f = pl.kernel(
        _gather_body,
        out_type=(jax.ShapeDtypeStruct((N_EDGES, DP), jnp.float32),
                  jax.ShapeDtypeStruct((N_EDGES, DP), jnp.float32)),
        mesh=mesh,
        scratch_types=[
            pltpu.VMEM((CHUNKS, C), jnp.int32),
            pltpu.VMEM((2, GE, DP), jnp.float32),
            pltpu.SemaphoreType.DMA,
            pltpu.SemaphoreType.DMA,
            pltpu.SemaphoreType.DMA,
        ],
        compiler_params=pltpu.CompilerParams(use_tc_tiling_on_sc=False),
    )
    return f(h, src2d, tgt2d)


def _sig(x):
    return 1.0 / (1.0 + jnp.exp(-x))


def _in_mlp_body(x_ref, w1_ref, b1_ref, w2_ref, b2_ref, o_ref):
    h1 = jnp.dot(x_ref[...], w1_ref[...], preferred_element_type=jnp.float32)
    h1 = h1 + b1_ref[...]
    h1 = h1 * _sig(h1)
    h2 = jnp.dot(h1, w2_ref[...], preferred_element_type=jnp.float32)
    o_ref[...] = h2 + b2_ref[...]


def _in_mlp(x, W1, b1, W2p, b2p):
    B = 2000
    grid = (N_NODES // B,)
    return pl.pallas_call(
        _in_mlp_body,
        grid=grid,
        in_specs=[
            pl.BlockSpec((B, IN_DIM), lambda i: (i, 0)),
            pl.BlockSpec((IN_DIM, HIDDEN), lambda i: (0, 0)),
            pl.BlockSpec((1, HIDDEN), lambda i: (0, 0)),
            pl.BlockSpec((HIDDEN, DP), lambda i: (0, 0)),
            pl.BlockSpec((1, DP), lambda i: (0, 0)),
        ],
        out_specs=pl.BlockSpec((B, DP), lambda i: (i, 0)),
        out_shape=jax.ShapeDtypeStruct((N_NODES, DP), jnp.float32),
    )(x, W1, b1, W2p, b2p)


NR = N_NODES // NS


def _scatter_body(h_hbm, m_hbm, tgt_hbm, parts_hbm,
                  idxbuf, mbuf, zbuf, acc, msem, ssem):
    c = lax.axis_index("c")
    s = lax.axis_index("s")
    wid = s * NC + c
    ebase = wid * EW

    def zloop(j, carry):
        zbuf[j, :] = jnp.zeros((DP,), jnp.float32)
        return carry

    lax.fori_loop(0, NR, zloop, 0)

    @pl.when(c == 0)
    def _():
        pltpu.sync_copy(h_hbm.at[pl.ds(s * NR, NR)], zbuf)

    pltpu.sync_copy(zbuf, acc.at[pl.ds(s * NR, NR)])
    pltpu.sync_copy(tgt_hbm.at[wid], idxbuf)
    plsc.subcore_barrier()

    ld = pltpu.async_copy(m_hbm.at[pl.ds(ebase, GE)], mbuf.at[0], msem)
    for g in range(NG):
        b = g & 1
        ld.wait()
        if g + 1 < NG:
            ld = pltpu.async_copy(
                m_hbm.at[pl.ds(ebase + (g + 1) * GE, GE)], mbuf.at[b ^ 1],
                msem)

        def fire(j, carry, g=g, b=b):
            pltpu.async_copy(mbuf.at[b, pl.ds(j * C, C)],
                             acc.at[idxbuf.at[g * GC + j]], ssem, add=True)
            return carry

        lax.fori_loop(0, GC, fire, 0)
        pltpu.make_async_copy(mbuf.at[b], acc.at[pl.ds(0, GE)], ssem).wait()

    plsc.subcore_barrier()

    pltpu.sync_copy(acc.at[pl.ds(s * NR, NR)],
                    parts_hbm.at[c, pl.ds(s * NR, NR)])


def _sc_scatter(h, m, tgt3d):
    mesh = plsc.VectorSubcoreMesh(core_axis_name="c", subcore_axis_name="s",
                                  num_cores=NC, num_subcores=NS)
    f = pl.kernel(
        _scatter_body,
        out_type=jax.ShapeDtypeStruct((NC, N_NODES, DP), jnp.float32),
        mesh=mesh,
        scratch_types=[
            pltpu.VMEM((CHUNKS, C), jnp.int32),
            pltpu.VMEM((2, GE, DP), jnp.float32),
            pltpu.VMEM((NR, DP), jnp.float32),
            pltpu.VMEM_SHARED((N_NODES, DP), jnp.float32),
            pltpu.SemaphoreType.DMA,
            pltpu.SemaphoreType.DMA,
        ],
        compiler_params=pltpu.CompilerParams(use_tc_tiling_on_sc=False),
    )
    return f(h, m, tgt3d)


def _add_body(a_ref, b_ref, o_ref):
    o_ref[...] = a_ref[0] + b_ref[0]


def _combine(parts):
    NPK = N_NODES * DP // 128
    return pl.pallas_call(
        _add_body,
        grid=(1,),
        in_specs=[
            pl.BlockSpec((1, NPK, 128), lambda i: (0, 0, 0)),
            pl.BlockSpec((1, NPK, 128), lambda i: (1, 0, 0)),
        ],
        out_specs=pl.BlockSpec((NPK, 128), lambda i: (0, 0)),
        out_shape=jax.ShapeDtypeStruct((NPK, 128), jnp.float32),
    )(parts, parts)


HID8 = 8 * HIDDEN
TRM8 = 8 * NNZ


def _edge_body(gs_ref, gt_ref, wbig_ref, b1_ref, wbr_ref, w2_ref, b2_ref,
               o_ref):
    x = jnp.concatenate([gs_ref[...], gt_ref[...]], axis=1)
    y = jnp.dot(x.astype(jnp.bfloat16), wbig_ref[...],
                preferred_element_type=jnp.float32)
    hid = y[:, :HID8]
    tt = y[:, HID8:HID8 + TRM8] * y[:, HID8 + TRM8:]
    hid = hid + jnp.dot(tt, wbr_ref[...], preferred_element_type=jnp.float32)
    hid = hid + b1_ref[...]
    hid = hid * _sig(hid)
    m = jnp.dot(hid.astype(jnp.bfloat16), w2_ref[...],
                preferred_element_type=jnp.float32)
    o_ref[...] = m + b2_ref[...]


def _edge_mlp(gs, gt, Wbig, b1, Wbr, W2p, b2p):
    B = 3200
    BP = B * DP // 128
    grid = (N_EDGES // B,)
    return pl.pallas_call(
        _edge_body,
        grid=grid,
        in_specs=[
            pl.BlockSpec((BP, 128), lambda i: (i, 0)),
            pl.BlockSpec((BP, 128), lambda i: (i, 0)),
            pl.BlockSpec((2 * 128, HID8 + 2 * TRM8), lambda i: (0, 0)),
            pl.BlockSpec((1, HID8), lambda i: (0, 0)),
            pl.BlockSpec((TRM8, HID8), lambda i: (0, 0)),
            pl.BlockSpec((HID8, 128), lambda i: (0, 0)),
            pl.BlockSpec((1, 128), lambda i: (0, 0)),
        ],
        out_specs=pl.BlockSpec((BP, 128), lambda i: (i, 0)),
        out_shape=jax.ShapeDtypeStruct((N_EDGES * DP // 128, 128),
                                       jnp.float32),
    )(gs, gt, Wbig.astype(jnp.bfloat16), b1, Wbr,
      W2p.astype(jnp.bfloat16), b2p)


def _pool_body(h_ref, k_ref, w1_ref, b1_ref, w2_ref, b2_ref, o_ref):
    hp = h_ref[...]
    sh = jnp.zeros((1, DP), jnp.float32)
    sk = jnp.zeros((1, 1), jnp.float32)
    for j in range(8):
        hj = hp[:, DP * j:DP * (j + 1)]
        hk = jnp.dot(hj, k_ref[...], preferred_element_type=jnp.float32)
        sk = sk + jnp.sum(hk * hj, keepdims=True).reshape(1, 1)
        sh = sh + jnp.sum(hj, axis=0, keepdims=True)
    ph = sh * (1.0 / N_NODES)
    pk = sk * (1.0 / N_NODES)
    p = jnp.concatenate([ph, pk, jnp.zeros((1, 2 * DP - DP - 1), jnp.float32)],
                        axis=1)
    z = jnp.dot(p, w1_ref[...], preferred_element_type=jnp.float32)
    z = z + b1_ref[...]
    z = z * _sig(z)
    o_ref[...] = jnp.dot(z, w2_ref[...],
                         preferred_element_type=jnp.float32) + b2_ref[...]


def _pool(h, Kp, Wo1p, bo1, Wo2, bo2):
    return pl.pallas_call(
        _pool_body,
        in_specs=[
            pl.BlockSpec((N_NODES * DP // 128, 128), lambda: (0, 0)),
            pl.BlockSpec((DP, DP), lambda: (0, 0)),
            pl.BlockSpec((2 * DP, HIDDEN), lambda: (0, 0)),
            pl.BlockSpec((1, HIDDEN), lambda: (0, 0)),
            pl.BlockSpec((HIDDEN, OUT_DIM), lambda: (0, 0)),
            pl.BlockSpec((1, OUT_DIM), lambda: (0, 0)),
        ],
        out_specs=pl.BlockSpec((1, OUT_DIM), lambda: (0, 0)),
        out_shape=jax.ShapeDtypeStruct((1, OUT_DIM), jnp.float32),
    )(h, Kp, Wo1p, bo1, Wo2, bo2)


def kernel(x, edge_index, W_in1, b_in1, W_in2, b_in2, l0_W1, l0_b1, l0_W2,
           l0_b2, l1_W1, l1_b1, l1_W2, l1_b2, Kmat, out_W1, out_b1, out_W2,
           out_b2, bI, bJ, bK, bC):
    src = edge_index[0]
    tgt = edge_index[1]

    def padr(a, r):
        return jnp.concatenate(
            [a, jnp.zeros((r - a.shape[0], a.shape[1]), a.dtype)], axis=0)

    def padc(a, c):
        return jnp.concatenate(
            [a, jnp.zeros((a.shape[0], c - a.shape[1]), a.dtype)], axis=1)

    W_in2p = padc(W_in2, DP)
    b_in2p = padc(b_in2[None, :], DP)

    zDN = jnp.zeros((DP, NNZ), jnp.float32)
    SI = (lax.broadcasted_iota(jnp.int32, (DP, NNZ), 0) ==
          bI[None, :]).astype(jnp.float32)
    SJ = (lax.broadcasted_iota(jnp.int32, (DP, NNZ), 0) ==
          bJ[None, :]).astype(jnp.float32)

    I8 = jnp.eye(8, dtype=jnp.float32)
    zBD = jnp.zeros((128, TRM8), jnp.float32)
    layers = []
    for (W1, b1, W2, b2) in ((l0_W1, l0_b1, l0_W2, l0_b2),
                             (l1_W1, l1_b1, l1_W2, l1_b2)):
        top = jnp.concatenate([jnp.kron(I8, padr(W1[:D_A], DP)),
                               jnp.kron(I8, SI), zBD], axis=1)
        bot = jnp.concatenate([jnp.kron(I8, padr(W1[D_A:2 * D_A], DP)),
                               zBD, jnp.kron(I8, SJ)], axis=1)
        Wbig = jnp.concatenate([top, bot], axis=0)
        Wbr = bC[:, None] * jnp.take(W1[2 * D_A:], bK, axis=0)
        WbrBD = jnp.kron(I8, Wbr)
        W2BD = jnp.kron(I8, padc(W2, DP))
        b1t = jnp.tile(b1[None, :], (1, 8))
        b2t = jnp.tile(padc(b2[None, :], DP), (1, 8))
        layers.append((Wbig, b1t, WbrBD, W2BD, b2t))

    Kp = padr(padc(Kmat, DP), DP)
    Wo1p = jnp.concatenate(
        [out_W1[:D_A], jnp.zeros((DP - D_A, HIDDEN), jnp.float32),
         out_W1[D_A:D_A + 1], jnp.zeros((2 * DP - DP - 1, HIDDEN),
                                        jnp.float32)], axis=0)

    src2d = src.reshape(NW, CHUNKS, C)
    tgt2d = tgt.reshape(NW, CHUNKS, C)
    hp = _in_mlp(x, W_in1, b_in1[None, :], W_in2p, b_in2p)

    EP = N_EDGES * DP // 128
    NPK = N_NODES * DP // 128
    for (Wbig, b1t, WbrBD, W2BD, b2t) in layers:
        hu = jnp.reshape(hp, (N_NODES, DP))
        gs, gt = _sc_gather(hu, src2d, tgt2d)
        m_p = _edge_mlp(jnp.reshape(gs, (EP, 128)),
                        jnp.reshape(gt, (EP, 128)),
                        Wbig, b1t, WbrBD, W2BD, b2t)
        parts = _sc_scatter(hu, jnp.reshape(m_p, (N_EDGES, DP)), tgt2d)
        hp = _combine(jnp.reshape(parts, (NC, NPK, 128)))

    return _pool(hp, Kp, Wo1p, out_b1[None, :], out_W2, out_b2[None, :])


# speedup vs baseline: 1.0393x; 1.0393x over previous
import functools
import jax
import jax.numpy as jnp
import numpy as np
from jax import lax
from jax.experimental import pallas as pl
from jax.experimental.pallas import tpu as pltpu
from jax.experimental.pallas import tpu_sc as plsc

N_NODES = 10000
N_EDGES = 320000
IN_DIM = 128
HIDDEN = 128
OUT_DIM = 32
D_A = 14
DP = 16
NNZ = 64

NC = 2
NS = 16
NW = NC * NS
NHALF = 2
EH = N_EDGES // NHALF
EW = EH // NW
C = 40
CHUNKS = EW // C
GC = 25
GE = GC * C
NG = CHUNKS // GC


def _gather_body(h_hbm, src_hbm, tgt_hbm, gs_hbm, gt_hbm,
                 idxbuf, rows, gsem, osem0, osem1):
    c = lax.axis_index("c")
    s = lax.axis_index("s")
    wid = s * NC + c
    ebase = wid * EW
    for idx_hbm, out_hbm in ((src_hbm, gs_hbm), (tgt_hbm, gt_hbm)):
        pltpu.sync_copy(idx_hbm.at[wid], idxbuf)
        osems = (osem0, osem1)
        descs = [None] * NG
        for g in range(NG):
            b = g & 1
            if g >= 2:
                descs[g - 2].wait()

            def fire(j, carry, g=g, b=b):
                pltpu.async_copy(h_hbm.at[idxbuf.at[g * GC + j]],
                                 rows.at[b, pl.ds(j * C, C)], gsem)
                return carry

            lax.fori_loop(0, GC, fire, 0)
            pltpu.make_async_copy(out_hbm.at[pl.ds(ebase + g * GE, GE)],
                                  rows.at[b], gsem).wait()
            descs[g] = pltpu.async_copy(
                rows.at[b], out_hbm.at[pl.ds(ebase + g * GE, GE)], osems[b])
        descs[NG - 2].wait()
        descs[NG - 1].wait()


def _sc_gather(h, src2d, tgt2d):
    mesh = plsc.VectorSubcoreMesh(core_axis_name="c", subcore_axis_name="s",
                                  num_cores=NC, num_subcores=NS)
    f = pl.kernel(
        _gather_body,
        out_type=(jax.ShapeDtypeStruct((EH, DP), jnp.float32),
                  jax.ShapeDtypeStruct((EH, DP), jnp.float32)),
        mesh=mesh,
        scratch_types=[
            pltpu.VMEM((CHUNKS, C), jnp.int32),
            pltpu.VMEM((2, GE, DP), jnp.float32),
            pltpu.SemaphoreType.DMA,
            pltpu.SemaphoreType.DMA,
            pltpu.SemaphoreType.DMA,
        ],
        compiler_params=pltpu.CompilerParams(use_tc_tiling_on_sc=False),
    )
    return f(h, src2d, tgt2d)


def _sig(x):
    return 1.0 / (1.0 + jnp.exp(-x))


def _in_mlp_body(x_ref, w1_ref, b1_ref, w2_ref, b2_ref, o_ref):
    h1 = jnp.dot(x_ref[...], w1_ref[...], preferred_element_type=jnp.float32)
    h1 = h1 + b1_ref[...]
    h1 = h1 * _sig(h1)
    h2 = jnp.dot(h1, w2_ref[...], preferred_element_type=jnp.float32)
    o_ref[...] = h2 + b2_ref[...]


def _in_mlp(x, W1, b1, W2p, b2p):
    B = 2000
    grid = (N_NODES // B,)
    return pl.pallas_call(
        _in_mlp_body,
        grid=grid,
        in_specs=[
            pl.BlockSpec((B, IN_DIM), lambda i: (i, 0)),
            pl.BlockSpec((IN_DIM, HIDDEN), lambda i: (0, 0)),
            pl.BlockSpec((1, HIDDEN), lambda i: (0, 0)),
            pl.BlockSpec((HIDDEN, DP), lambda i: (0, 0)),
            pl.BlockSpec((1, DP), lambda i: (0, 0)),
        ],
        out_specs=pl.BlockSpec((B, DP), lambda i: (i, 0)),
        out_shape=jax.ShapeDtypeStruct((N_NODES, DP), jnp.float32),
    )(x, W1, b1, W2p, b2p)


NR = N_NODES // NS


def _scatter_body(h_hbm, m_hbm, tgt_hbm, parts_hbm,
                  idxbuf, mbuf, zbuf, acc, msem, ssem, seed=True):
    c = lax.axis_index("c")
    s = lax.axis_index("s")
    wid = s * NC + c
    ebase = wid * EW

    def zloop(j, carry):
        zbuf[j, :] = jnp.zeros((DP,), jnp.float32)
        return carry

    lax.fori_loop(0, NR, zloop, 0)

    if seed:
        @pl.when(c == 0)
        def _():
            pltpu.sync_copy(h_hbm.at[pl.ds(s * NR, NR)], zbuf)

    pltpu.sync_copy(zbuf, acc.at[pl.ds(s * NR, NR)])
    pltpu.sync_copy(tgt_hbm.at[wid], idxbuf)
    plsc.subcore_barrier()

    ld = pltpu.async_copy(m_hbm.at[pl.ds(ebase, GE)], mbuf.at[0], msem)
    for g in range(NG):
        b = g & 1
        ld.wait()
        if g + 1 < NG:
            ld = pltpu.async_copy(
                m_hbm.at[pl.ds(ebase + (g + 1) * GE, GE)], mbuf.at[b ^ 1],
                msem)

        def fire(j, carry, g=g, b=b):
            pltpu.async_copy(mbuf.at[b, pl.ds(j * C, C)],
                             acc.at[idxbuf.at[g * GC + j]], ssem, add=True)
            return carry

        lax.fori_loop(0, GC, fire, 0)
        pltpu.make_async_copy(mbuf.at[b], acc.at[pl.ds(0, GE)], ssem).wait()

    plsc.subcore_barrier()

    pltpu.sync_copy(acc.at[pl.ds(s * NR, NR)],
                    parts_hbm.at[c, pl.ds(s * NR, NR)])


def _sc_scatter(h, m, tgt3d, seed):
    mesh = plsc.VectorSubcoreMesh(core_axis_name="c", subcore_axis_name="s",
                                  num_cores=NC, num_subcores=NS)
    f = pl.kernel(
        functools.partial(_scatter_body, seed=seed),
        out_type=jax.ShapeDtypeStruct((NC, N_NODES, DP), jnp.float32),
        mesh=mesh,
        scratch_types=[
            pltpu.VMEM((CHUNKS, C), jnp.int32),
            pltpu.VMEM((2, GE, DP), jnp.float32),
            pltpu.VMEM((NR, DP), jnp.float32),
            pltpu.VMEM_SHARED((N_NODES, DP), jnp.float32),
            pltpu.SemaphoreType.DMA,
            pltpu.SemaphoreType.DMA,
        ],
        compiler_params=pltpu.CompilerParams(use_tc_tiling_on_sc=False),
    )
    return f(h, m, tgt3d)


def _add_body(a_ref, b_ref, c_ref, d_ref, o_ref):
    o_ref[...] = (a_ref[0] + b_ref[0]) + (c_ref[0] + d_ref[0])


def _combine(p0, p1):
    NPK = N_NODES * DP // 128
    return pl.pallas_call(
        _add_body,
        grid=(1,),
        in_specs=[
            pl.BlockSpec((1, NPK, 128), lambda i: (0, 0, 0)),
            pl.BlockSpec((1, NPK, 128), lambda i: (1, 0, 0)),
            pl.BlockSpec((1, NPK, 128), lambda i: (0, 0, 0)),
            pl.BlockSpec((1, NPK, 128), lambda i: (1, 0, 0)),
        ],
        out_specs=pl.BlockSpec((NPK, 128), lambda i: (0, 0)),
        out_shape=jax.ShapeDtypeStruct((NPK, 128), jnp.float32),
    )(p0, p0, p1, p1)


HID8 = 8 * HIDDEN
TRM8 = 8 * NNZ


def _edge_body(gs_ref, gt_ref, wbig_ref, b1_ref, wbr_ref, w2_ref, b2_ref,
               o_ref):
    x = jnp.concatenate([gs_ref[...], gt_ref[...]], axis=1)
    y = jnp.dot(x, wbig_ref[...], preferred_element_type=jnp.float32)
    hid = y[:, :HID8]
    tt = y[:, HID8:HID8 + TRM8] * y[:, HID8 + TRM8:]
    hid = hid + jnp.dot(tt, wbr_ref[...], preferred_element_type=jnp.float32)
    hid = hid + b1_ref[...]
    hid = hid * _sig(hid)
    m = jnp.dot(hid, w2_ref[...], preferred_element_type=jnp.float32)
    o_ref[...] = m + b2_ref[...]


def _edge_mlp(gs, gt, Wbig, b1, Wbr, W2p, b2p):
    B = 3200
    BP = B * DP // 128
    grid = (EH // B,)
    return pl.pallas_call(
        _edge_body,
        grid=grid,
        in_specs=[
            pl.BlockSpec((BP, 128), lambda i: (i, 0)),
            pl.BlockSpec((BP, 128), lambda i: (i, 0)),
            pl.BlockSpec((2 * 128, HID8 + 2 * TRM8), lambda i: (0, 0)),
            pl.BlockSpec((1, HID8), lambda i: (0, 0)),
            pl.BlockSpec((TRM8, HID8), lambda i: (0, 0)),
            pl.BlockSpec((HID8, 128), lambda i: (0, 0)),
            pl.BlockSpec((1, 128), lambda i: (0, 0)),
        ],
        out_specs=pl.BlockSpec((BP, 128), lambda i: (i, 0)),
        out_shape=jax.ShapeDtypeStruct((EH * DP // 128, 128),
                                       jnp.float32),
    )(gs, gt, Wbig, b1, Wbr, W2p, b2p)


def _pool_body(h_ref, k_ref, w1_ref, b1_ref, w2_ref, b2_ref, o_ref):
    hp = h_ref[...]
    sh = jnp.zeros((1, DP), jnp.float32)
    sk = jnp.zeros((1, 1), jnp.float32)
    for j in range(8):
        hj = hp[:, DP * j:DP * (j + 1)]
        hk = jnp.dot(hj, k_ref[...], preferred_element_type=jnp.float32)
        sk = sk + jnp.sum(hk * hj, keepdims=True).reshape(1, 1)
        sh = sh + jnp.sum(hj, axis=0, keepdims=True)
    ph = sh * (1.0 / N_NODES)
    pk = sk * (1.0 / N_NODES)
    p = jnp.concatenate([ph, pk, jnp.zeros((1, 2 * DP - DP - 1), jnp.float32)],
                        axis=1)
    z = jnp.dot(p, w1_ref[...], preferred_element_type=jnp.float32)
    z = z + b1_ref[...]
    z = z * _sig(z)
    o_ref[...] = jnp.dot(z, w2_ref[...],
                         preferred_element_type=jnp.float32) + b2_ref[...]


def _pool(h, Kp, Wo1p, bo1, Wo2, bo2):
    return pl.pallas_call(
        _pool_body,
        in_specs=[
            pl.BlockSpec((N_NODES * DP // 128, 128), lambda: (0, 0)),
            pl.BlockSpec((DP, DP), lambda: (0, 0)),
            pl.BlockSpec((2 * DP, HIDDEN), lambda: (0, 0)),
            pl.BlockSpec((1, HIDDEN), lambda: (0, 0)),
            pl.BlockSpec((HIDDEN, OUT_DIM), lambda: (0, 0)),
            pl.BlockSpec((1, OUT_DIM), lambda: (0, 0)),
        ],
        out_specs=pl.BlockSpec((1, OUT_DIM), lambda: (0, 0)),
        out_shape=jax.ShapeDtypeStruct((1, OUT_DIM), jnp.float32),
    )(h, Kp, Wo1p, bo1, Wo2, bo2)


def kernel(x, edge_index, W_in1, b_in1, W_in2, b_in2, l0_W1, l0_b1, l0_W2,
           l0_b2, l1_W1, l1_b1, l1_W2, l1_b2, Kmat, out_W1, out_b1, out_W2,
           out_b2, bI, bJ, bK, bC):
    src = edge_index[0]
    tgt = edge_index[1]

    def padr(a, r):
        return jnp.concatenate(
            [a, jnp.zeros((r - a.shape[0], a.shape[1]), a.dtype)], axis=0)

    def padc(a, c):
        return jnp.concatenate(
            [a, jnp.zeros((a.shape[0], c - a.shape[1]), a.dtype)], axis=1)

    W_in2p = padc(W_in2, DP)
    b_in2p = padc(b_in2[None, :], DP)

    zDN = jnp.zeros((DP, NNZ), jnp.float32)
    SI = (lax.broadcasted_iota(jnp.int32, (DP, NNZ), 0) ==
          bI[None, :]).astype(jnp.float32)
    SJ = (lax.broadcasted_iota(jnp.int32, (DP, NNZ), 0) ==
          bJ[None, :]).astype(jnp.float32)

    I8 = jnp.eye(8, dtype=jnp.float32)
    zBD = jnp.zeros((128, TRM8), jnp.float32)
    layers = []
    for (W1, b1, W2, b2) in ((l0_W1, l0_b1, l0_W2, l0_b2),
                             (l1_W1, l1_b1, l1_W2, l1_b2)):
        top = jnp.concatenate([jnp.kron(I8, padr(W1[:D_A], DP)),
                               jnp.kron(I8, SI), zBD], axis=1)
        bot = jnp.concatenate([jnp.kron(I8, padr(W1[D_A:2 * D_A], DP)),
                               zBD, jnp.kron(I8, SJ)], axis=1)
        Wbig = jnp.concatenate([top, bot], axis=0)
        Wbr = bC[:, None] * jnp.take(W1[2 * D_A:], bK, axis=0)
        WbrBD = jnp.kron(I8, Wbr)
        W2BD = jnp.kron(I8, padc(W2, DP))
        b1t = jnp.tile(b1[None, :], (1, 8))
        b2t = jnp.tile(padc(b2[None, :], DP), (1, 8))
        layers.append((Wbig, b1t, WbrBD, W2BD, b2t))

    Kp = padr(padc(Kmat, DP), DP)
    Wo1p = jnp.concatenate(
        [out_W1[:D_A], jnp.zeros((DP - D_A, HIDDEN), jnp.float32),
         out_W1[D_A:D_A + 1], jnp.zeros((2 * DP - DP - 1, HIDDEN),
                                        jnp.float32)], axis=0)

    src3 = src.reshape(NHALF, NW, CHUNKS, C)
    tgt3 = tgt.reshape(NHALF, NW, CHUNKS, C)
    hp = _in_mlp(x, W_in1, b_in1[None, :], W_in2p, b_in2p)

    EP = EH * DP // 128
    NPK = N_NODES * DP // 128
    for (Wbig, b1t, WbrBD, W2BD, b2t) in layers:
        hu = jnp.reshape(hp, (N_NODES, DP))
        parts = []
        ms = [None, None]
        for half in range(NHALF):
            gs, gt = _sc_gather(hu, src3[half], tgt3[half])
            ms[half] = _edge_mlp(jnp.reshape(gs, (EP, 128)),
                                 jnp.reshape(gt, (EP, 128)),
                                 Wbig, b1t, WbrBD, W2BD, b2t)
        for half in range(NHALF):
            parts.append(_sc_scatter(hu, jnp.reshape(ms[half], (EH, DP)),
                                     tgt3[half], seed=(half == 0)))
        hp = _combine(jnp.reshape(parts[0], (NC, NPK, 128)),
                      jnp.reshape(parts[1], (NC, NPK, 128)))

    return _pool(hp, Kp, Wo1p, out_b1[None, :], out_W2, out_b2[None, :])


# speedup vs baseline: 1.1372x; 1.0942x over previous
import functools
import jax
import jax.numpy as jnp
import numpy as np
from jax import lax
from jax.experimental import pallas as pl
from jax.experimental.pallas import tpu as pltpu
from jax.experimental.pallas import tpu_sc as plsc

N_NODES = 10000
N_EDGES = 320000
IN_DIM = 128
HIDDEN = 128
OUT_DIM = 32
D_A = 14
DP = 16
NNZ = 64

NC = 2
NS = 16
NW = NC * NS
NHALF = 2
EH = N_EDGES // NHALF
EW = EH // NW
C = 40
CHUNKS = EW // C
GC = 25
GE = GC * C
NG = CHUNKS // GC


def _gather_body(h_hbm, src_hbm, tgt_hbm, gs_hbm, gt_hbm,
                 idxbuf, rows, gsem, osem0, osem1):
    c = lax.axis_index("c")
    s = lax.axis_index("s")
    wid = s * NC + c
    ebase = wid * EW
    for idx_hbm, out_hbm in ((src_hbm, gs_hbm), (tgt_hbm, gt_hbm)):
        pltpu.sync_copy(idx_hbm.at[wid], idxbuf)
        osems = (osem0, osem1)
        descs = [None] * NG
        for g in range(NG):
            b = g & 1
            if g >= 2:
                descs[g - 2].wait()

            def fire(j, carry, g=g, b=b):
                pltpu.async_copy(h_hbm.at[idxbuf.at[g * GC + j]],
                                 rows.at[b, pl.ds(j * C, C)], gsem)
                return carry

            lax.fori_loop(0, GC, fire, 0)
            pltpu.make_async_copy(out_hbm.at[pl.ds(ebase + g * GE, GE)],
                                  rows.at[b], gsem).wait()
            descs[g] = pltpu.async_copy(
                rows.at[b], out_hbm.at[pl.ds(ebase + g * GE, GE)], osems[b])
        descs[NG - 2].wait()
        descs[NG - 1].wait()


def _sc_gather(h, src2d, tgt2d):
    mesh = plsc.VectorSubcoreMesh(core_axis_name="c", subcore_axis_name="s",
                                  num_cores=NC, num_subcores=NS)
    f = pl.kernel(
        _gather_body,
        out_type=(jax.ShapeDtypeStruct((EH, DP), jnp.float32),
                  jax.ShapeDtypeStruct((EH, DP), jnp.float32)),
        mesh=mesh,
        scratch_types=[
            pltpu.VMEM((CHUNKS, C), jnp.int32),
            pltpu.VMEM((2, GE, DP), jnp.float32),
            pltpu.SemaphoreType.DMA,
            pltpu.SemaphoreType.DMA,
            pltpu.SemaphoreType.DMA,
        ],
        compiler_params=pltpu.CompilerParams(use_tc_tiling_on_sc=False),
    )
    return f(h, src2d, tgt2d)


def _sig(x):
    return 1.0 / (1.0 + jnp.exp(-x))


def _in_mlp_body(x_ref, w1_ref, b1_ref, w2_ref, b2_ref, o_ref):
    h1 = jnp.dot(x_ref[...], w1_ref[...], preferred_element_type=jnp.float32)
    h1 = h1 + b1_ref[...]
    h1 = h1 * _sig(h1)
    h2 = jnp.dot(h1, w2_ref[...], preferred_element_type=jnp.float32)
    o_ref[...] = h2 + b2_ref[...]


def _in_mlp(x, W1, b1, W2p, b2p):
    B = 2000
    grid = (N_NODES // B,)
    return pl.pallas_call(
        _in_mlp_body,
        grid=grid,
        in_specs=[
            pl.BlockSpec((B, IN_DIM), lambda i: (i, 0)),
            pl.BlockSpec((IN_DIM, HIDDEN), lambda i: (0, 0)),
            pl.BlockSpec((1, HIDDEN), lambda i: (0, 0)),
            pl.BlockSpec((HIDDEN, DP), lambda i: (0, 0)),
            pl.BlockSpec((1, DP), lambda i: (0, 0)),
        ],
        out_specs=pl.BlockSpec((B, DP), lambda i: (i, 0)),
        out_shape=jax.ShapeDtypeStruct((N_NODES, DP), jnp.float32),
    )(x, W1, b1, W2p, b2p)


NR = N_NODES // NS


def _scatter_body(h_hbm, m_hbm, tgt_hbm, parts_hbm,
                  idxbuf, mbuf, zbuf, acc, msem, ssem, seed=True):
    c = lax.axis_index("c")
    s = lax.axis_index("s")
    wid = s * NC + c
    ebase = wid * EW

    def zloop(j, carry):
        zbuf[j, :] = jnp.zeros((DP,), jnp.float32)
        return carry

    lax.fori_loop(0, NR, zloop, 0)

    if seed:
        @pl.when(c == 0)
        def _():
            pltpu.sync_copy(h_hbm.at[pl.ds(s * NR, NR)], zbuf)

    pltpu.sync_copy(zbuf, acc.at[pl.ds(s * NR, NR)])
    pltpu.sync_copy(tgt_hbm.at[wid], idxbuf)
    plsc.subcore_barrier()

    ld = pltpu.async_copy(m_hbm.at[pl.ds(ebase, GE)], mbuf.at[0], msem)
    for g in range(NG):
        b = g & 1
        ld.wait()
        if g + 1 < NG:
            ld = pltpu.async_copy(
                m_hbm.at[pl.ds(ebase + (g + 1) * GE, GE)], mbuf.at[b ^ 1],
                msem)

        def fire(j, carry, g=g, b=b):
            pltpu.async_copy(mbuf.at[b, pl.ds(j * C, C)],
                             acc.at[idxbuf.at[g * GC + j]], ssem, add=True)
            return carry

        lax.fori_loop(0, GC, fire, 0)
        pltpu.make_async_copy(mbuf.at[b], acc.at[pl.ds(0, GE)], ssem).wait()

    plsc.subcore_barrier()

    pltpu.sync_copy(acc.at[pl.ds(s * NR, NR)],
                    parts_hbm.at[c, pl.ds(s * NR, NR)])


def _sc_scatter(h, m, tgt3d, seed):
    mesh = plsc.VectorSubcoreMesh(core_axis_name="c", subcore_axis_name="s",
                                  num_cores=NC, num_subcores=NS)
    f = pl.kernel(
        functools.partial(_scatter_body, seed=seed),
        out_type=jax.ShapeDtypeStruct((NC, N_NODES, DP), jnp.float32),
        mesh=mesh,
        scratch_types=[
            pltpu.VMEM((CHUNKS, C), jnp.int32),
            pltpu.VMEM((2, GE, DP), jnp.float32),
            pltpu.VMEM((NR, DP), jnp.float32),
            pltpu.VMEM_SHARED((N_NODES, DP), jnp.float32),
            pltpu.SemaphoreType.DMA,
            pltpu.SemaphoreType.DMA,
        ],
        compiler_params=pltpu.CompilerParams(use_tc_tiling_on_sc=False),
    )
    return f(h, m, tgt3d)


def _add_body(a_ref, b_ref, c_ref, d_ref, o_ref):
    o_ref[...] = (a_ref[0] + b_ref[0]) + (c_ref[0] + d_ref[0])


def _combine(p0, p1):
    NPK = N_NODES * DP // 128
    return pl.pallas_call(
        _add_body,
        grid=(1,),
        in_specs=[
            pl.BlockSpec((1, NPK, 128), lambda i: (0, 0, 0)),
            pl.BlockSpec((1, NPK, 128), lambda i: (1, 0, 0)),
            pl.BlockSpec((1, NPK, 128), lambda i: (0, 0, 0)),
            pl.BlockSpec((1, NPK, 128), lambda i: (1, 0, 0)),
        ],
        out_specs=pl.BlockSpec((NPK, 128), lambda i: (0, 0)),
        out_shape=jax.ShapeDtypeStruct((NPK, 128), jnp.float32),
    )(p0, p0, p1, p1)


HID8 = 8 * HIDDEN
TRM8 = 8 * NNZ


def _edge_body(gs_ref, gt_ref, wbig_ref, b1_ref, wbr_ref, w2_ref, b2_ref,
               o_ref):
    x = jnp.concatenate([gs_ref[...], gt_ref[...]], axis=1)
    y = jnp.dot(x, wbig_ref[...], preferred_element_type=jnp.float32)
    hid = y[:, :HID8]
    tt = y[:, HID8:HID8 + TRM8] * y[:, HID8 + TRM8:]
    hid = hid + jnp.dot(tt, wbr_ref[...], preferred_element_type=jnp.float32)
    hid = hid + b1_ref[...]
    hid = hid * _sig(hid)
    m = jnp.dot(hid, w2_ref[...], preferred_element_type=jnp.float32)
    o_ref[...] = m + b2_ref[...]


def _edge_mlp(gs, gt, Wbig, b1, Wbr, W2p, b2p):
    B = 6400
    BP = B * DP // 128
    grid = (EH // B,)
    return pl.pallas_call(
        _edge_body,
        grid=grid,
        in_specs=[
            pl.BlockSpec((BP, 128), lambda i: (i, 0)),
            pl.BlockSpec((BP, 128), lambda i: (i, 0)),
            pl.BlockSpec((2 * 128, HID8 + 2 * TRM8), lambda i: (0, 0)),
            pl.BlockSpec((1, HID8), lambda i: (0, 0)),
            pl.BlockSpec((TRM8, HID8), lambda i: (0, 0)),
            pl.BlockSpec((HID8, 128), lambda i: (0, 0)),
            pl.BlockSpec((1, 128), lambda i: (0, 0)),
        ],
        out_specs=pl.BlockSpec((BP, 128), lambda i: (i, 0)),
        out_shape=jax.ShapeDtypeStruct((EH * DP // 128, 128),
                                       jnp.float32),
    )(gs, gt, Wbig, b1, Wbr, W2p, b2p)


def _pool_body(h_ref, k_ref, w1_ref, b1_ref, w2_ref, b2_ref, o_ref):
    hp = h_ref[...]
    sh = jnp.zeros((1, DP), jnp.float32)
    sk = jnp.zeros((1, 1), jnp.float32)
    for j in range(8):
        hj = hp[:, DP * j:DP * (j + 1)]
        hk = jnp.dot(hj, k_ref[...], preferred_element_type=jnp.float32)
        sk = sk + jnp.sum(hk * hj, keepdims=True).reshape(1, 1)
        sh = sh + jnp.sum(hj, axis=0, keepdims=True)
    ph = sh * (1.0 / N_NODES)
    pk = sk * (1.0 / N_NODES)
    p = jnp.concatenate([ph, pk, jnp.zeros((1, 2 * DP - DP - 1), jnp.float32)],
                        axis=1)
    z = jnp.dot(p, w1_ref[...], preferred_element_type=jnp.float32)
    z = z + b1_ref[...]
    z = z * _sig(z)
    o_ref[...] = jnp.dot(z, w2_ref[...],
                         preferred_element_type=jnp.float32) + b2_ref[...]


def _pool(h, Kp, Wo1p, bo1, Wo2, bo2):
    return pl.pallas_call(
        _pool_body,
        in_specs=[
            pl.BlockSpec((N_NODES * DP // 128, 128), lambda: (0, 0)),
            pl.BlockSpec((DP, DP), lambda: (0, 0)),
            pl.BlockSpec((2 * DP, HIDDEN), lambda: (0, 0)),
            pl.BlockSpec((1, HIDDEN), lambda: (0, 0)),
            pl.BlockSpec((HIDDEN, OUT_DIM), lambda: (0, 0)),
            pl.BlockSpec((1, OUT_DIM), lambda: (0, 0)),
        ],
        out_specs=pl.BlockSpec((1, OUT_DIM), lambda: (0, 0)),
        out_shape=jax.ShapeDtypeStruct((1, OUT_DIM), jnp.float32),
    )(h, Kp, Wo1p, bo1, Wo2, bo2)


def kernel(x, edge_index, W_in1, b_in1, W_in2, b_in2, l0_W1, l0_b1, l0_W2,
           l0_b2, l1_W1, l1_b1, l1_W2, l1_b2, Kmat, out_W1, out_b1, out_W2,
           out_b2, bI, bJ, bK, bC):
    src = edge_index[0]
    tgt = edge_index[1]

    def padr(a, r):
        return jnp.concatenate(
            [a, jnp.zeros((r - a.shape[0], a.shape[1]), a.dtype)], axis=0)

    def padc(a, c):
        return jnp.concatenate(
            [a, jnp.zeros((a.shape[0], c - a.shape[1]), a.dtype)], axis=1)

    W_in2p = padc(W_in2, DP)
    b_in2p = padc(b_in2[None, :], DP)

    zDN = jnp.zeros((DP, NNZ), jnp.float32)
    SI = (lax.broadcasted_iota(jnp.int32, (DP, NNZ), 0) ==
          bI[None, :]).astype(jnp.float32)
    SJ = (lax.broadcasted_iota(jnp.int32, (DP, NNZ), 0) ==
          bJ[None, :]).astype(jnp.float32)

    I8 = jnp.eye(8, dtype=jnp.float32)
    zBD = jnp.zeros((128, TRM8), jnp.float32)
    layers = []
    for (W1, b1, W2, b2) in ((l0_W1, l0_b1, l0_W2, l0_b2),
                             (l1_W1, l1_b1, l1_W2, l1_b2)):
        top = jnp.concatenate([jnp.kron(I8, padr(W1[:D_A], DP)),
                               jnp.kron(I8, SI), zBD], axis=1)
        bot = jnp.concatenate([jnp.kron(I8, padr(W1[D_A:2 * D_A], DP)),
                               zBD, jnp.kron(I8, SJ)], axis=1)
        Wbig = jnp.concatenate([top, bot], axis=0)
        Wbr = bC[:, None] * jnp.take(W1[2 * D_A:], bK, axis=0)
        WbrBD = jnp.kron(I8, Wbr)
        W2BD = jnp.kron(I8, padc(W2, DP))
        b1t = jnp.tile(b1[None, :], (1, 8))
        b2t = jnp.tile(padc(b2[None, :], DP), (1, 8))
        layers.append((Wbig, b1t, WbrBD, W2BD, b2t))

    Kp = padr(padc(Kmat, DP), DP)
    Wo1p = jnp.concatenate(
        [out_W1[:D_A], jnp.zeros((DP - D_A, HIDDEN), jnp.float32),
         out_W1[D_A:D_A + 1], jnp.zeros((2 * DP - DP - 1, HIDDEN),
                                        jnp.float32)], axis=0)

    src3 = src.reshape(NHALF, NW, CHUNKS, C)
    tgt3 = tgt.reshape(NHALF, NW, CHUNKS, C)
    hp = _in_mlp(x, W_in1, b_in1[None, :], W_in2p, b_in2p)

    EP = EH * DP // 128
    NPK = N_NODES * DP // 128
    for (Wbig, b1t, WbrBD, W2BD, b2t) in layers:
        hu = jnp.reshape(hp, (N_NODES, DP))
        parts = []
        ms = [None, None]
        for half in range(NHALF):
            gs, gt = _sc_gather(hu, src3[half], tgt3[half])
            ms[half] = _edge_mlp(jnp.reshape(gs, (EP, 128)),
                                 jnp.reshape(gt, (EP, 128)),
                                 Wbig, b1t, WbrBD, W2BD, b2t)
        for half in range(NHALF):
            parts.append(_sc_scatter(hu, jnp.reshape(ms[half], (EH, DP)),
                                     tgt3[half], seed=(half == 0)))
        hp = _combine(jnp.reshape(parts[0], (NC, NPK, 128)),
                      jnp.reshape(parts[1], (NC, NPK, 128)))

    return _pool(hp, Kp, Wo1p, out_b1[None, :], out_W2, out_b2[None, :])
